# baseline (speedup 1.0000x reference)
import functools

import jax
import jax.numpy as jnp
from jax import lax
from jax.experimental import pallas as pl
from jax.experimental.pallas import tpu as pltpu
from jax.experimental.pallas import tpu_sc as plsc

N = 10000
E = 320000
D = 128
NP = 10240

NC = 2
NS = 16
NW = NC * NS
EPW = E // NW
EPT = E // NS
K = 40
NCHUNK = EPW // K
NB = 5
NG = NCHUNK // NB
RPT = NP // NS
NH = 128

_mesh = plsc.VectorSubcoreMesh(
    core_axis_name="c", subcore_axis_name="s", num_cores=NC, num_subcores=NS
)


def _rsqrt_nr(d):
    i = plsc.bitcast(d, jnp.int32)
    i = jnp.int32(0x5F3759DF) - (i >> 1)
    y = plsc.bitcast(i, jnp.float32)
    for _ in range(3):
        y = y * (1.5 - 0.5 * d * y * y)
    return y


@functools.partial(
    pl.kernel,
    out_type=jax.ShapeDtypeStruct((2, NH, 128), jnp.float32),
    mesh=_mesh,
    scratch_types=[
        pltpu.VMEM((EPT,), jnp.int32),
        pltpu.VMEM((EPT,), jnp.int32),
        pltpu.VMEM((NH, 128), jnp.float32),
        pltpu.VMEM((NH, 128), jnp.float32),
        pltpu.VMEM((NH,), jnp.int32),
        pltpu.VMEM_SHARED((NH, 128), jnp.float32),
        pltpu.VMEM_SHARED((NH, 128), jnp.float32),
    ],
    compiler_params=pltpu.CompilerParams(needs_layout_passes=False),
)
def _deg_kernel(src_hbm, dst_hbm, z_hbm, iota_hbm, out_hbm,
                sidx, didx, h_o, h_i, irows, s_o, s_i):
    cid = lax.axis_index("c")
    sid = lax.axis_index("s")
    r0 = sid * (NH // NS)
    pltpu.sync_copy(z_hbm, h_o)
    pltpu.sync_copy(z_hbm, h_i)
    pltpu.sync_copy(iota_hbm, irows)
    pltpu.sync_copy(src_hbm.at[pl.ds(sid * EPT, EPT)], sidx)
    pltpu.sync_copy(dst_hbm.at[pl.ds(sid * EPT, EPT)], didx)
    pltpu.sync_copy(z_hbm.at[pl.ds(r0, NH // NS)], s_o.at[pl.ds(r0, NH // NS)])
    pltpu.sync_copy(z_hbm.at[pl.ds(r0, NH // NS)], s_i.at[pl.ds(r0, NH // NS)])

    ones = jnp.ones((16,), jnp.float32)

    @pl.loop(0, EPT // 16)
    def _vec(j):
        iv = sidx[pl.ds(j * 16, 16)]
        plsc.addupdate_scatter(h_o, [iv >> 7, iv & 127], ones)
        dv = didx[pl.ds(j * 16, 16)]
        plsc.addupdate_scatter(h_i, [dv >> 7, dv & 127], ones)

    plsc.subcore_barrier()
    pltpu.sync_copy(h_o, s_o.at[irows], add=True)
    pltpu.sync_copy(h_i, s_i.at[irows], add=True)
    plsc.subcore_barrier()

    @pl.when(sid < 8)
    def _norms():
        rd = cid * (NH // NC) + sid * 8
        pltpu.sync_copy(s_o.at[pl.ds(rd, 8)], h_o.at[pl.ds(0, 8)])
        pltpu.sync_copy(s_i.at[pl.ds(rd, 8)], h_i.at[pl.ds(0, 8)])
        for r in range(8):
            for v in range(8):
                sl = pl.ds(v * 16, 16)
                h_o[r, sl] = _rsqrt_nr(jnp.maximum(h_o[r, sl], 1.0))
                h_i[r, sl] = _rsqrt_nr(jnp.maximum(h_i[r, sl], 1.0))
        pltpu.sync_copy(h_o.at[pl.ds(0, 8)], out_hbm.at[0, pl.ds(rd, 8)])
        pltpu.sync_copy(h_i.at[pl.ds(0, 8)], out_hbm.at[1, pl.ds(rd, 8)])


@functools.partial(
    pl.kernel,
    out_type=jax.ShapeDtypeStruct((NC, NP, D), jnp.float32),
    mesh=_mesh,
    scratch_types=[
        pltpu.VMEM((EPW,), jnp.int32),
        pltpu.VMEM_SHARED((NP, D), jnp.float32),
    ]
    + [pltpu.VMEM((K, D), jnp.float32) for _ in range(NB)]
    + [pltpu.VMEM((K,), jnp.int32) for _ in range(NB)]
    + [pltpu.SemaphoreType.DMA for _ in range(2 * NB)],
)
def _agg_kernel(h_hbm, src_hbm, dst_hbm, z_hbm, out_hbm,
                sidx, acc, *rest):
    rows = rest[:NB]
    didx = rest[NB:2 * NB]
    gsem = rest[2 * NB:3 * NB]
    isem = rest[3 * NB:]
    cid = lax.axis_index("c")
    sid = lax.axis_index("s")
    wid = sid * NC + cid
    r0 = sid * RPT
    ebase = wid * EPW
    pltpu.sync_copy(src_hbm.at[pl.ds(ebase, EPW)], sidx)
    pltpu.sync_copy(z_hbm.at[pl.ds(r0, RPT)], acc.at[pl.ds(r0, RPT)])

    for b in range(NB):
        pltpu.async_copy(dst_hbm.at[pl.ds(ebase + b * K, K)], didx[b], isem[b])
        pltpu.async_copy(h_hbm.at[sidx.at[pl.ds(b * K, K)]], rows[b], gsem[b])
    plsc.subcore_barrier()

    @pl.loop(0, NG - 1)
    def _group(g):
        c0 = g * NB
        for b in range(NB):
            pltpu.make_async_copy(h_hbm.at[pl.ds(0, K)], rows[b], gsem[b]).wait()
            pltpu.make_async_copy(dst_hbm.at[pl.ds(0, K)], didx[b], isem[b]).wait()
            pltpu.sync_copy(rows[b], acc.at[didx[b]], add=True)
            nxt = (c0 + b + NB) * K
            pltpu.async_copy(dst_hbm.at[pl.ds(ebase + nxt, K)], didx[b], isem[b])
            pltpu.async_copy(h_hbm.at[sidx.at[pl.ds(nxt, K)]], rows[b], gsem[b])

    for b in range(NB):
        pltpu.make_async_copy(h_hbm.at[pl.ds(0, K)], rows[b], gsem[b]).wait()
        pltpu.make_async_copy(dst_hbm.at[pl.ds(0, K)], didx[b], isem[b]).wait()
        pltpu.sync_copy(rows[b], acc.at[didx[b]], add=True)

    plsc.subcore_barrier()
    pltpu.sync_copy(acc.at[pl.ds(r0, RPT)], out_hbm.at[cid, pl.ds(r0, RPT)])


B = 400


def _pre0_body(x_ref, no_ref, w_ref, o_ref):
    o_ref[...] = jnp.dot(x_ref[...] * no_ref[...], w_ref[...],
                         preferred_element_type=jnp.float32)


def _mid_body(a_ref, ni_ref, no_ref, b_ref, w_ref, o_ref):
    a = a_ref[0] + a_ref[1]
    h = jax.nn.sigmoid(a * ni_ref[...] + b_ref[...])
    o_ref[...] = jnp.dot(h * no_ref[...], w_ref[...],
                         preferred_element_type=jnp.float32)


def _final_body(a_ref, ni_ref, b_ref, o_ref):
    a = a_ref[0] + a_ref[1]
    o_ref[...] = jax.nn.sigmoid(a * ni_ref[...] + b_ref[...])


_acc_spec = pl.BlockSpec((NC, B, D), lambda i: (0, i, 0))
_row_spec = pl.BlockSpec((B, D), lambda i: (i, 0))
_w_spec = pl.BlockSpec((D, D), lambda i: (0, 0))
_b_spec = pl.BlockSpec((1, D), lambda i: (0, 0))
_out_t = jax.ShapeDtypeStruct((N, D), jnp.float32)
_grid = (N // B,)

_pre0 = pl.pallas_call(
    _pre0_body, grid=_grid,
    in_specs=[_row_spec, _row_spec, _w_spec],
    out_specs=_row_spec, out_shape=_out_t)

_mid = pl.pallas_call(
    _mid_body, grid=_grid,
    in_specs=[_acc_spec, _row_spec, _row_spec, _b_spec, _w_spec],
    out_specs=_row_spec, out_shape=_out_t)

_final = pl.pallas_call(
    _final_body, grid=_grid,
    in_specs=[_acc_spec, _row_spec, _b_spec],
    out_specs=_row_spec, out_shape=_out_t)


def kernel(x, edge_index, W0, b0, W1, b1, W2, b2):
    src = edge_index[0]
    dst = edge_index[1]
    zh = jnp.zeros((NH, 128), jnp.float32)
    iota = jnp.arange(NH, dtype=jnp.int32)
    z128 = jnp.zeros((NP, D), jnp.float32)
    b0 = b0.reshape(1, D)
    b1 = b1.reshape(1, D)
    b2 = b2.reshape(1, D)

    norms = _deg_kernel(src, dst, zh, iota)
    n_o = jnp.broadcast_to(norms[0].reshape(NH * 128)[:N, None], (N, D))
    n_i = jnp.broadcast_to(norms[1].reshape(NH * 128)[:N, None], (N, D))

    h = _pre0(x, n_o, W0)
    a = _agg_kernel(h, src, dst, z128)
    h = _mid(a, n_i, n_o, b0, W1)
    a = _agg_kernel(h, src, dst, z128)
    h = _mid(a, n_i, n_o, b1, W2)
    a = _agg_kernel(h, src, dst, z128)
    return _final(a, n_i, b2)


# speedup vs baseline: 1.1009x; 1.1009x over previous
import functools

import jax
import jax.numpy as jnp
from jax import lax
from jax.experimental import pallas as pl
from jax.experimental.pallas import tpu as pltpu
from jax.experimental.pallas import tpu_sc as plsc

N = 10000
E = 320000
D = 128
NP = 10240

NC = 2
NS = 16
NW = NC * NS
EPW = E // NW
K = 40
NCHUNK = EPW // K
NB = 5
NG = NCHUNK // NB
RPT = NP // NS
NH = 128

_mesh = plsc.VectorSubcoreMesh(
    core_axis_name="c", subcore_axis_name="s", num_cores=NC, num_subcores=NS
)


HRPT = NH // NS


@functools.partial(
    pl.kernel,
    out_type=jax.ShapeDtypeStruct((NC, 2, NH, 128), jnp.float32),
    mesh=_mesh,
    scratch_types=[
        pltpu.VMEM((EPW,), jnp.int32),
        pltpu.VMEM((EPW,), jnp.int32),
        pltpu.VMEM((NH, 128), jnp.float32),
        pltpu.VMEM((NH, 128), jnp.float32),
        pltpu.VMEM((NH,), jnp.int32),
        pltpu.VMEM_SHARED((NH, 128), jnp.float32),
        pltpu.VMEM_SHARED((NH, 128), jnp.float32),
    ],
    compiler_params=pltpu.CompilerParams(needs_layout_passes=False,
                                         skip_device_barrier=True),
)
def _deg_kernel(src_hbm, dst_hbm, z_hbm, iota_hbm, out_hbm,
                sidx, didx, h_o, h_i, irows, s_o, s_i):
    cid = lax.axis_index("c")
    sid = lax.axis_index("s")
    wid = sid * NC + cid
    r0 = sid * HRPT
    pltpu.sync_copy(z_hbm, h_o)
    pltpu.sync_copy(z_hbm, h_i)
    pltpu.sync_copy(iota_hbm, irows)
    pltpu.sync_copy(src_hbm.at[pl.ds(wid * EPW, EPW)], sidx)
    pltpu.sync_copy(dst_hbm.at[pl.ds(wid * EPW, EPW)], didx)
    pltpu.sync_copy(z_hbm.at[pl.ds(r0, HRPT)], s_o.at[pl.ds(r0, HRPT)])
    pltpu.sync_copy(z_hbm.at[pl.ds(r0, HRPT)], s_i.at[pl.ds(r0, HRPT)])

    ones = jnp.ones((16,), jnp.float32)

    @pl.loop(0, EPW // 16)
    def _vec(j):
        iv = sidx[pl.ds(j * 16, 16)]
        plsc.addupdate_scatter(h_o, [iv >> 7, iv & 127], ones)
        dv = didx[pl.ds(j * 16, 16)]
        plsc.addupdate_scatter(h_i, [dv >> 7, dv & 127], ones)

    plsc.subcore_barrier()
    pltpu.sync_copy(h_o, s_o.at[irows], add=True)
    pltpu.sync_copy(h_i, s_i.at[irows], add=True)
    plsc.subcore_barrier()
    pltpu.sync_copy(s_o.at[pl.ds(r0, HRPT)], out_hbm.at[cid, 0, pl.ds(r0, HRPT)])
    pltpu.sync_copy(s_i.at[pl.ds(r0, HRPT)], out_hbm.at[cid, 1, pl.ds(r0, HRPT)])


@functools.partial(
    pl.kernel,
    out_type=jax.ShapeDtypeStruct((NC, NP, D), jnp.float32),
    mesh=_mesh,
    scratch_types=[
        pltpu.VMEM((EPW,), jnp.int32),
        pltpu.VMEM_SHARED((NP, D), jnp.float32),
    ]
    + [pltpu.VMEM((K, D), jnp.float32) for _ in range(NB)]
    + [pltpu.VMEM((K,), jnp.int32) for _ in range(NB)]
    + [pltpu.SemaphoreType.DMA for _ in range(2 * NB)],
    compiler_params=pltpu.CompilerParams(skip_device_barrier=True),
)
def _agg_kernel(h_hbm, src_hbm, dst_hbm, z_hbm, out_hbm,
                sidx, acc, *rest):
    rows = rest[:NB]
    didx = rest[NB:2 * NB]
    gsem = rest[2 * NB:3 * NB]
    isem = rest[3 * NB:]
    cid = lax.axis_index("c")
    sid = lax.axis_index("s")
    wid = sid * NC + cid
    r0 = sid * RPT
    ebase = wid * EPW
    pltpu.sync_copy(src_hbm.at[pl.ds(ebase, EPW)], sidx)
    pltpu.sync_copy(z_hbm.at[pl.ds(r0, RPT)], acc.at[pl.ds(r0, RPT)])

    for b in range(NB):
        pltpu.async_copy(dst_hbm.at[pl.ds(ebase + b * K, K)], didx[b], isem[b])
        pltpu.async_copy(h_hbm.at[sidx.at[pl.ds(b * K, K)]], rows[b], gsem[b])
    plsc.subcore_barrier()

    @pl.loop(0, NG - 1)
    def _group(g):
        c0 = g * NB
        for b in range(NB):
            pltpu.make_async_copy(h_hbm.at[pl.ds(0, K)], rows[b], gsem[b]).wait()
            pltpu.make_async_copy(dst_hbm.at[pl.ds(0, K)], didx[b], isem[b]).wait()
            pltpu.sync_copy(rows[b], acc.at[didx[b]], add=True)
            nxt = (c0 + b + NB) * K
            pltpu.async_copy(dst_hbm.at[pl.ds(ebase + nxt, K)], didx[b], isem[b])
            pltpu.async_copy(h_hbm.at[sidx.at[pl.ds(nxt, K)]], rows[b], gsem[b])

    for b in range(NB):
        pltpu.make_async_copy(h_hbm.at[pl.ds(0, K)], rows[b], gsem[b]).wait()
        pltpu.make_async_copy(dst_hbm.at[pl.ds(0, K)], didx[b], isem[b]).wait()
        pltpu.sync_copy(rows[b], acc.at[didx[b]], add=True)

    plsc.subcore_barrier()
    pltpu.sync_copy(acc.at[pl.ds(r0, RPT)], out_hbm.at[cid, pl.ds(r0, RPT)])


B = 2048


def _norm_body(degs_ref, o_ref):
    d_o = degs_ref[0, 0] + degs_ref[1, 0]
    d_i = degs_ref[0, 1] + degs_ref[1, 1]
    o_ref[0] = lax.rsqrt(jnp.maximum(d_o, 1.0))
    o_ref[1] = lax.rsqrt(jnp.maximum(d_i, 1.0))


_norm = pl.pallas_call(
    _norm_body,
    out_shape=jax.ShapeDtypeStruct((2, NH, 128), jnp.float32))


def _pre0_body(x_ref, no_ref, w_ref, o_ref):
    o_ref[...] = jnp.dot(x_ref[...] * no_ref[...], w_ref[...],
                         preferred_element_type=jnp.float32)


def _mid_body(a_ref, ni_ref, no_ref, b_ref, w_ref, o_ref):
    a = a_ref[0] + a_ref[1]
    h = jax.nn.sigmoid(a * ni_ref[...] + b_ref[...])
    o_ref[...] = jnp.dot(h * no_ref[...], w_ref[...],
                         preferred_element_type=jnp.float32)


def _final_body(a_ref, ni_ref, b_ref, o_ref):
    a = a_ref[0] + a_ref[1]
    o_ref[...] = jax.nn.sigmoid(a * ni_ref[...] + b_ref[...])


_acc_spec = pl.BlockSpec((NC, B, D), lambda i: (0, i, 0))
_row_spec = pl.BlockSpec((B, D), lambda i: (i, 0))
_w_spec = pl.BlockSpec((D, D), lambda i: (0, 0))
_b_spec = pl.BlockSpec((1, D), lambda i: (0, 0))
_out_t = jax.ShapeDtypeStruct((NP, D), jnp.float32)
_grid = (NP // B,)

_pre0 = pl.pallas_call(
    _pre0_body, grid=_grid,
    in_specs=[_row_spec, _row_spec, _w_spec],
    out_specs=_row_spec, out_shape=_out_t)

_mid = pl.pallas_call(
    _mid_body, grid=_grid,
    in_specs=[_acc_spec, _row_spec, _row_spec, _b_spec, _w_spec],
    out_specs=_row_spec, out_shape=_out_t)

_final = pl.pallas_call(
    _final_body, grid=_grid,
    in_specs=[_acc_spec, _row_spec, _b_spec],
    out_specs=_row_spec, out_shape=_out_t)


def kernel(x, edge_index, W0, b0, W1, b1, W2, b2):
    src = edge_index[0]
    dst = edge_index[1]
    xp = jnp.pad(x, ((0, NP - N), (0, 0)))
    zh = jnp.zeros((NH, 128), jnp.float32)
    iota = jnp.arange(NH, dtype=jnp.int32)
    z128 = jnp.zeros((NP, D), jnp.float32)
    b0 = b0.reshape(1, D)
    b1 = b1.reshape(1, D)
    b2 = b2.reshape(1, D)

    degs = _deg_kernel(src, dst, zh, iota)
    norms = _norm(degs)
    n_o = jnp.broadcast_to(norms[0].reshape(NH * 128)[:NP, None], (NP, D))
    n_i = jnp.broadcast_to(norms[1].reshape(NH * 128)[:NP, None], (NP, D))

    h = _pre0(xp, n_o, W0)
    a = _agg_kernel(h, src, dst, z128)
    h = _mid(a, n_i, n_o, b0, W1)
    a = _agg_kernel(h, src, dst, z128)
    h = _mid(a, n_i, n_o, b1, W2)
    a = _agg_kernel(h, src, dst, z128)
    return _final(a, n_i, b2)[:N]
